# Initial kernel scaffold; baseline (speedup 1.0000x reference)
import jax
import jax.numpy as jnp
from jax.experimental import pallas as pl


def kernel(inputs, m_list, targets):
    raise NotImplementedError("write your pallas kernel here")


# speedup vs baseline: 8.8724x; 8.8724x over previous
import jax
import jax.numpy as jnp
from jax.experimental import pallas as pl
from jax.experimental.pallas import tpu as pltpu

_S = 30.0


def _ldam_block(x_ref, t_ref, m_ref, out_ref):
    i = pl.program_id(0)
    nb = pl.num_programs(0)
    x = x_ref[...]
    t = t_ref[...]
    m_row = m_ref[...]
    bm, c = x.shape
    col = jax.lax.broadcasted_iota(jnp.int32, (bm, c), 1)
    mask = col == t
    batch_m = jnp.sum(jnp.where(mask, m_row, 0.0), axis=1, keepdims=True)
    w = _S * (x - jnp.where(mask, batch_m, 0.0))
    mx = jnp.max(w, axis=1, keepdims=True)
    s = jnp.sum(jnp.exp(w - mx), axis=1)
    wt = jnp.sum(jnp.where(mask, w, 0.0), axis=1)
    blk = jnp.sum(jnp.log(s) + mx[:, 0] - wt)

    @pl.when(i == 0)
    def _init():
        out_ref[0, 0] = 0.0

    out_ref[0, 0] += blk

    @pl.when(i == nb - 1)
    def _fin():
        out_ref[0, 0] = out_ref[0, 0] / (bm * nb)


def kernel(inputs, m_list, targets):
    n, c = inputs.shape
    bm = 2048
    grid = (n // bm,)
    out = pl.pallas_call(
        _ldam_block,
        grid=grid,
        in_specs=[
            pl.BlockSpec((bm, c), lambda i: (i, 0)),
            pl.BlockSpec((bm, 1), lambda i: (i, 0)),
            pl.BlockSpec((1, c), lambda i: (0, 0)),
        ],
        out_specs=pl.BlockSpec(
            (1, 1), lambda i: (0, 0), memory_space=pltpu.SMEM
        ),
        out_shape=jax.ShapeDtypeStruct((1, 1), jnp.float32),
    )(inputs, targets.reshape(n, 1), m_list.reshape(1, c))
    return out[0, 0]
